# baseline (speedup 1.0000x reference)
import functools

import jax
import jax.numpy as jnp
from jax import lax
from jax.experimental import pallas as pl
from jax.experimental.pallas import tpu as pltpu
from jax.experimental.pallas import tpu_sc as plsc

_BLK = 128
_BPC = 10
_LANES = 16


@functools.lru_cache(maxsize=None)
def _build_gather(vocab, emb, nb, nt):
    info = plsc.get_sparse_core_info()
    nc, ns = info.num_cores, info.num_subcores
    nw = nc * ns
    batch = nb * nt
    chunk = _BPC * _BLK
    nblk = nb // _BLK
    b_per_w = batch // nw
    n_chunks = b_per_w // chunk
    assert n_chunks % 2 == 0
    eg = emb // 8
    blk_f = 8 * _BLK
    mesh = plsc.VectorSubcoreMesh(core_axis_name="c", subcore_axis_name="s")

    @functools.partial(
        pl.kernel,
        mesh=mesh,
        out_type=jax.ShapeDtypeStruct((batch * emb,), jnp.float32),
        scratch_types=[
            pltpu.VMEM((2, chunk), jnp.int32),
            pltpu.VMEM((2, chunk, emb), jnp.float32),
            pltpu.VMEM((2, emb * _BLK), jnp.float32),
            pltpu.SemaphoreType.DMA,
            pltpu.SemaphoreType.DMA,
        ],
        compiler_params=pltpu.CompilerParams(
            use_tc_tiling_on_sc=False, needs_layout_passes=False
        ),
    )
    def k(table_hbm, idx_hbm, out_hbm, idx_v, rows_v, piece_v, gsem, wsem):
        wid = lax.axis_index("s") * nc + lax.axis_index("c")
        base = wid * b_per_w
        lane = lax.iota(jnp.int32, _LANES)
        idx_ls = [lane + l0 for l0 in range(0, _BLK, _LANES)]

        def start_gather(j, slot):
            pltpu.sync_copy(
                idx_hbm.at[pl.ds(base + j * chunk, chunk)], idx_v.at[slot]
            )
            pltpu.async_copy(table_hbm.at[idx_v.at[slot]], rows_v.at[slot], gsem)

        def wait_gather(slot):
            pltpu.make_async_copy(
                table_hbm.at[idx_v.at[slot]], rows_v.at[slot], gsem
            ).wait()

        def each_piece(j, i, pslot):
            m = j * _BPC + i
            gw = base // _BLK + m
            t = gw // nblk
            bb = gw - t * nblk
            for g in range(eg):
                src = piece_v.at[pslot, pl.ds(g * blk_f, blk_f)]
                dst = out_hbm.at[pl.ds(((t * eg + g) * nblk + bb) * blk_f, blk_f)]
                yield src, dst

        def wait_writeback(j, i, pslot):
            for src, dst in each_piece(j, i, pslot):
                pltpu.make_async_copy(src, dst, wsem).wait()

        def start_writeback(j, i, pslot):
            for src, dst in each_piece(j, i, pslot):
                pltpu.async_copy(src, dst, wsem)

        def transpose_block(i, slot, pslot):
            l_base = i * _BLK

            def e_body(e4, carry):
                for u in range(4):
                    e = e4 * 4 + u
                    idx_e = jnp.full((_LANES,), e, jnp.int32)
                    d0 = e * _BLK
                    for li, l0 in enumerate(range(0, _BLK, _LANES)):
                        idx_l = idx_ls[li] + l_base
                        v = plsc.load_gather(rows_v.at[slot], [idx_l, idx_e])
                        piece_v[pslot, pl.ds(d0 + l0, _LANES)] = v
                return carry

            lax.fori_loop(0, emb // 4, e_body, 0)

        start_gather(0, 0)

        def body(j2, carry):
            for b in range(2):
                j = 2 * j2 + b
                slot = b
                if b == 0:
                    start_gather(j + 1, 1)
                else:
                    @pl.when(j2 < (n_chunks // 2) - 1)
                    def _():
                        start_gather(j + 1, 0)
                wait_gather(slot)
                for i in range(_BPC):
                    pslot = i % 2
                    if i >= 2:
                        wait_writeback(j, i - 2, pslot)
                    else:
                        @pl.when(j > 0)
                        def _(j=j, i=i, pslot=pslot):
                            wait_writeback(j - 1, _BPC + i - 2, pslot)

                    transpose_block(i, slot, pslot)
                    start_writeback(j, i, pslot)
            return carry

        lax.fori_loop(0, n_chunks // 2, body, 0)
        wait_writeback(n_chunks - 1, _BPC - 2, 0)
        wait_writeback(n_chunks - 1, _BPC - 1, 1)

    return k


def _tc_w_body(x_ref, o_ref):
    x = x_ref[...]
    o_ref[...] = x.reshape(32, 256, 4).transpose(1, 2, 0).reshape(256, 128)


@functools.lru_cache(maxsize=None)
def _build_w_rows(vocab, emb):
    cols = 1024
    grid = (vocab + cols - 1) // cols
    rows = vocab // 4
    return pl.pallas_call(
        _tc_w_body,
        grid=(grid,),
        in_specs=[pl.BlockSpec((emb, cols), lambda i: (0, i))],
        out_specs=pl.BlockSpec((cols // 4, 128), lambda i: (i, 0)),
        out_shape=jax.ShapeDtypeStruct((rows, 128), jnp.float32),
    )


def kernel(tokens, W):
    nb, nt = tokens.shape
    emb = W.shape[1]
    batch = nb * nt
    w_rows = _build_w_rows(W.shape[0], emb)(W.T).reshape(W.shape)
    flat = tokens.T.reshape(batch).astype(jnp.int32)
    outf = _build_gather(W.shape[0], emb, nb, nt)(w_rows, flat)
    out5 = outf.reshape(nt, emb // 8, nb // 128, 8, 128)
    out = out5.transpose(2, 4, 0, 1, 3)
    return out.reshape(nb, nt, emb)


# speedup vs baseline: 3.1956x; 3.1956x over previous
import functools

import jax
import jax.numpy as jnp
from jax import lax
from jax.experimental import pallas as pl
from jax.experimental.pallas import tpu as pltpu
from jax.experimental.pallas import tpu_sc as plsc

_BLK = 128
_BPC = 10
_LANES = 16


@functools.lru_cache(maxsize=None)
def _build_gather(vocab, emb, nb, nt):
    info = plsc.get_sparse_core_info()
    nc, ns = info.num_cores, info.num_subcores
    nw = nc * ns
    batch = nb * nt
    chunk = _BPC * _BLK
    nblk = nb // _BLK
    b_per_w = batch // nw
    n_chunks = b_per_w // chunk
    assert n_chunks % 2 == 0
    eg = emb // 8
    blk_f = 8 * _BLK
    mesh = plsc.VectorSubcoreMesh(core_axis_name="c", subcore_axis_name="s")

    @functools.partial(
        pl.kernel,
        mesh=mesh,
        out_type=jax.ShapeDtypeStruct((batch * emb,), jnp.float32),
        scratch_types=[
            pltpu.VMEM((2, chunk), jnp.int32),
            pltpu.VMEM((2, chunk, emb), jnp.float32),
            pltpu.VMEM((2, emb * _BLK + 64), jnp.float32),
            pltpu.SemaphoreType.DMA,
            pltpu.SemaphoreType.DMA,
        ],
        compiler_params=pltpu.CompilerParams(
            use_tc_tiling_on_sc=False, needs_layout_passes=False
        ),
    )
    def k(table_hbm, idx_hbm, out_hbm, idx_v, rows_v, piece_v, gsem, wsem):
        wid = lax.axis_index("s") * nc + lax.axis_index("c")
        base = wid * b_per_w
        lane = lax.iota(jnp.int32, _LANES)
        sc_lo = [lane * _BLK + dl for dl in range(8)]
        sc_hi = [(lane + _LANES) * _BLK + dl for dl in range(8)]

        def start_gather(j, slot):
            pltpu.sync_copy(
                idx_hbm.at[pl.ds(base + j * chunk, chunk)], idx_v.at[slot]
            )
            pltpu.async_copy(table_hbm.at[idx_v.at[slot]], rows_v.at[slot], gsem)

        def wait_gather(slot):
            pltpu.make_async_copy(
                table_hbm.at[idx_v.at[slot]], rows_v.at[slot], gsem
            ).wait()

        def each_piece(j, i, pslot):
            m = j * _BPC + i
            gw = base // _BLK + m
            t = gw // nblk
            bb = gw - t * nblk
            for g in range(eg):
                src = piece_v.at[pslot, pl.ds(g * blk_f, blk_f)]
                dst = out_hbm.at[pl.ds(((t * eg + g) * nblk + bb) * blk_f, blk_f)]
                yield src, dst

        def wait_writeback(j, i, pslot):
            for src, dst in each_piece(j, i, pslot):
                pltpu.make_async_copy(src, dst, wsem).wait()

        def start_writeback(j, i, pslot):
            for src, dst in each_piece(j, i, pslot):
                pltpu.async_copy(src, dst, wsem)

        def transpose_block(i, slot, pslot):
            l_base = i * _BLK
            span = (emb - 1) * _BLK + 8

            def l8_body(l8, carry):
                lg = l_base + l8 * 8
                dst = piece_v.at[pslot, pl.ds(l8 * 8, span)]
                for dl in range(8):
                    a = rows_v[slot, lg + dl, pl.ds(0, _LANES)]
                    b = rows_v[slot, lg + dl, pl.ds(_LANES, _LANES)]
                    plsc.store_scatter(dst, [sc_lo[dl]], a)
                    plsc.store_scatter(dst, [sc_hi[dl]], b)
                return carry

            lax.fori_loop(0, _BLK // 8, l8_body, 0)

        start_gather(0, 0)

        def body(j2, carry):
            for b in range(2):
                j = 2 * j2 + b
                slot = b
                if b == 0:
                    start_gather(j + 1, 1)
                else:
                    @pl.when(j2 < (n_chunks // 2) - 1)
                    def _():
                        start_gather(j + 1, 0)
                wait_gather(slot)
                for i in range(_BPC):
                    pslot = i % 2
                    if i >= 2:
                        wait_writeback(j, i - 2, pslot)
                    else:
                        @pl.when(j > 0)
                        def _(j=j, i=i, pslot=pslot):
                            wait_writeback(j - 1, _BPC + i - 2, pslot)

                    transpose_block(i, slot, pslot)
                    start_writeback(j, i, pslot)
            return carry

        lax.fori_loop(0, n_chunks // 2, body, 0)
        wait_writeback(n_chunks - 1, _BPC - 2, 0)
        wait_writeback(n_chunks - 1, _BPC - 1, 1)

    return k


def kernel(tokens, W):
    nb, nt = tokens.shape
    emb = W.shape[1]
    batch = nb * nt
    flat = tokens.T.reshape(batch).astype(jnp.int32)
    outf = _build_gather(W.shape[0], emb, nb, nt)(W, flat)
    out5 = outf.reshape(nt, emb // 8, nb // 128, 8, 128)
    out = out5.transpose(2, 4, 0, 1, 3)
    return out.reshape(nb, nt, emb)


# speedup vs baseline: 4.1109x; 1.2864x over previous
import functools

import jax
import jax.numpy as jnp
from jax import lax
from jax.experimental import pallas as pl
from jax.experimental.pallas import tpu as pltpu
from jax.experimental.pallas import tpu_sc as plsc

_BLK = 128
_BPC = 10
_LANES = 16


@functools.lru_cache(maxsize=None)
def _build_gather(vocab, emb, nb, nt):
    info = plsc.get_sparse_core_info()
    nc, ns = info.num_cores, info.num_subcores
    nw = nc * ns
    batch = nb * nt
    chunk = _BPC * _BLK
    nblk = nb // _BLK
    b_per_w = batch // nw
    n_chunks = b_per_w // chunk
    assert n_chunks % 2 == 0
    eg = emb // 8
    blk_f = 8 * _BLK
    mesh = plsc.VectorSubcoreMesh(core_axis_name="c", subcore_axis_name="s")

    @functools.partial(
        pl.kernel,
        mesh=mesh,
        out_type=jax.ShapeDtypeStruct((batch * emb // _BLK, _BLK), jnp.float32),
        scratch_types=[
            pltpu.VMEM((2, chunk), jnp.int32),
            pltpu.VMEM((2, chunk, emb), jnp.float32),
            pltpu.VMEM((2, emb, 137), jnp.float32),
            pltpu.SemaphoreType.DMA,
            pltpu.SemaphoreType.DMA,
        ],
        compiler_params=pltpu.CompilerParams(
            use_tc_tiling_on_sc=False, needs_layout_passes=False
        ),
    )
    def k(table_hbm, idx_hbm, out_hbm, idx_v, rows_v, piece_v, gsem, wsem):
        wid = lax.axis_index("s") * nc + lax.axis_index("c")
        base = wid * b_per_w
        lane = lax.iota(jnp.int32, _LANES)
        lane_hi = lane + _LANES

        def start_gather(j, slot):
            pltpu.sync_copy(
                idx_hbm.at[pl.ds(base + j * chunk, chunk)], idx_v.at[slot]
            )
            pltpu.async_copy(table_hbm.at[idx_v.at[slot]], rows_v.at[slot], gsem)

        def wait_gather(slot):
            pltpu.make_async_copy(
                table_hbm.at[idx_v.at[slot]], rows_v.at[slot], gsem
            ).wait()

        def each_piece(j, i, pslot):
            m = j * _BPC + i
            gw = base // _BLK + m
            t = gw // nblk
            bb = gw - t * nblk
            for g in range(eg):
                src = piece_v.at[pslot, pl.ds(g * 8, 8), pl.ds(0, _BLK)]
                dst = out_hbm.at[pl.ds(((t * eg + g) * nblk + bb) * 8, 8), :]
                yield src, dst

        def wait_writeback(j, i, pslot):
            for src, dst in each_piece(j, i, pslot):
                pltpu.make_async_copy(src, dst, wsem).wait()

        def start_writeback(j, i, pslot):
            for src, dst in each_piece(j, i, pslot):
                pltpu.async_copy(src, dst, wsem)

        def transpose_block(i, slot, pslot):
            l_base = i * _BLK
            dst = piece_v.at[pslot]

            def l4_body(l4, carry):
                for dl in range(4):
                    l = l4 * 4 + dl
                    lg = l_base + l
                    a = rows_v[slot, lg, pl.ds(0, _LANES)]
                    b = rows_v[slot, lg, pl.ds(_LANES, _LANES)]
                    idx_l = jnp.full((_LANES,), l, jnp.int32)
                    plsc.store_scatter(dst, [lane, idx_l], a)
                    plsc.store_scatter(dst, [lane_hi, idx_l], b)
                return carry

            lax.fori_loop(0, _BLK // 4, l4_body, 0)

        start_gather(0, 0)

        def body(j2, carry):
            for b in range(2):
                j = 2 * j2 + b
                slot = b
                if b == 0:
                    start_gather(j + 1, 1)
                else:
                    @pl.when(j2 < (n_chunks // 2) - 1)
                    def _():
                        start_gather(j + 1, 0)
                wait_gather(slot)
                for i in range(_BPC):
                    pslot = i % 2
                    if i >= 2:
                        wait_writeback(j, i - 2, pslot)
                    else:
                        @pl.when(j > 0)
                        def _(j=j, i=i, pslot=pslot):
                            wait_writeback(j - 1, _BPC + i - 2, pslot)

                    transpose_block(i, slot, pslot)
                    start_writeback(j, i, pslot)
            return carry

        lax.fori_loop(0, n_chunks // 2, body, 0)
        wait_writeback(n_chunks - 1, _BPC - 2, 0)
        wait_writeback(n_chunks - 1, _BPC - 1, 1)

    return k


def kernel(tokens, W):
    nb, nt = tokens.shape
    emb = W.shape[1]
    batch = nb * nt
    flat = tokens.T.reshape(batch).astype(jnp.int32)
    outf = _build_gather(W.shape[0], emb, nb, nt)(W, flat)
    out5 = outf.reshape(nt, emb // 8, nb // 128, 8, 128)
    out = out5.transpose(2, 4, 0, 1, 3)
    return out.reshape(nb, nt, emb)
